# baseline (speedup 1.0000x reference)
import jax
import jax.numpy as jnp
from jax.experimental import pallas as pl

_T = 4
_H = 12
_NB = 2


def _body(x_ref, wq_ref, wkv_ref, we_ref, out_ref):
    C = x_ref.shape[1]
    N = x_ref.shape[0] // _NB
    hd = C // _H
    R = _NB * _T * _H
    NE = _NB * _T
    M = _NB * N
    scale = hd ** -0.5

    blk = x_ref[...]
    wk = wkv_ref[:C, :]
    wv = wkv_ref[C:, :]

    q_t = []
    for t in range(_T):
        xt = jnp.concatenate(
            [x_ref[bi * N + t:bi * N + t + 1, :] for bi in range(_NB)], axis=0)
        q_t.append(jax.lax.dot_general(xt, wq_ref[t],
                                       (((1,), (1,)), ((), ()))))
    k = jax.lax.dot_general(blk, wk, (((1,), (1,)), ((), ())))

    r_iota = jax.lax.broadcasted_iota(jnp.int32, (R, C), 0)
    c_iota = jax.lax.broadcasted_iota(jnp.int32, (R, C), 1)
    hmask = (r_iota % _H) == (c_iota // hd)

    q96 = jnp.concatenate(
        [jnp.broadcast_to(q_t[t][bi:bi + 1, :], (_H, C))
         for bi in range(_NB) for t in range(_T)], axis=0)
    qm = jnp.where(hmask, q96, 0.0)
    scores = jax.lax.dot_general(qm, k, (((1,), (1,)), ((), ()))) * scale

    n_iota = jax.lax.broadcasted_iota(jnp.int32, (R, M), 1)
    rr_iota = jax.lax.broadcasted_iota(jnp.int32, (R, M), 0)
    neg_big = jnp.float32(-3.4e38)
    valid = ((n_iota // N) == (rr_iota // (_T * _H))) & ((n_iota % N) >= _T)
    scores = jnp.where(valid, scores, neg_big)

    m1 = jnp.max(scores, axis=1, keepdims=True)
    idx1 = jnp.min(jnp.where(scores == m1, n_iota, M), axis=1, keepdims=True)
    masked = jnp.where(n_iota == idx1, neg_big, scores)
    m2 = jnp.max(masked, axis=1, keepdims=True)
    idx2 = jnp.min(jnp.where(masked == m2, n_iota, M), axis=1, keepdims=True)

    e2 = jnp.exp(m2 - m1)
    den = 1.0 + e2
    w1 = 1.0 / den
    w2 = e2 / den

    s1 = jnp.where(n_iota == idx1, 1.0, 0.0)
    s2 = jnp.where(n_iota == idx2, 1.0, 0.0)
    d1 = s1 * w1
    d2 = s2 * w2

    g1 = jax.lax.dot_general(d1, blk, (((1,), (0,)), ((), ())))
    g2 = jax.lax.dot_general(d2, blk, (((1,), (0,)), ((), ())))
    gm1 = jnp.where(hmask, g1, 0.0)
    gm2 = jnp.where(hmask, g2, 0.0)

    v = jax.lax.dot_general(g1 + g2, wv, (((1,), (1,)), ((), ())))
    vm = jnp.where(hmask, v, 0.0)
    attn = vm.reshape(NE, _H, C).sum(axis=1)

    tok_t = []
    c_parts = []
    for t in range(_T):
        we_t = we_ref[t]
        at = jnp.concatenate(
            [attn[bi * _T + t:bi * _T + t + 1, :] for bi in range(_NB)], axis=0)
        tok_t.append(jax.lax.dot_general(at, we_t,
                                         (((1,), (1,)), ((), ()))))
        gm_cat = jnp.concatenate(
            [gm1[bi * _T * _H + t * _H:bi * _T * _H + (t + 1) * _H, :]
             for bi in range(_NB)] +
            [gm2[bi * _T * _H + t * _H:bi * _T * _H + (t + 1) * _H, :]
             for bi in range(_NB)], axis=0)
        c_parts.append(jax.lax.dot_general(gm_cat, we_t,
                                           (((1,), (1,)), ((), ()))))

    HH = _NB * _H
    c1 = jnp.concatenate(
        [c_parts[t][bi * _H:(bi + 1) * _H, :]
         for bi in range(_NB) for t in range(_T)], axis=0)
    c2 = jnp.concatenate(
        [c_parts[t][HH + bi * _H:HH + (bi + 1) * _H, :]
         for bi in range(_NB) for t in range(_T)], axis=0)
    tok = jnp.concatenate(tok_t, axis=0)

    en_iota = jax.lax.broadcasted_iota(jnp.int32, (NE, M), 1)
    er_iota = jax.lax.broadcasted_iota(jnp.int32, (NE, M), 0)
    e = jnp.where(en_iota == (er_iota % _NB) * N + er_iota // _NB, 1.0, 0.0)

    out_ref[...] = (
        jax.lax.dot_general(s1, c1, (((0,), (0,)), ((), ()))) +
        jax.lax.dot_general(s2, c2, (((0,), (0,)), ((), ()))) +
        jax.lax.dot_general(e, tok, (((0,), (0,)), ((), ()))))


def kernel(x, Wq, Wkv, We):
    B, N, C = x.shape
    x2 = x.reshape(B * N, C)
    G = B // _NB
    MB = _NB * N

    out = pl.pallas_call(
        _body,
        grid=(G,),
        in_specs=[
            pl.BlockSpec((MB, C), lambda g: (g, 0)),
            pl.BlockSpec((_T, C, C), lambda g: (0, 0, 0)),
            pl.BlockSpec((2 * C, C), lambda g: (0, 0)),
            pl.BlockSpec((_T, C, C), lambda g: (0, 0, 0)),
        ],
        out_specs=pl.BlockSpec((MB, C), lambda g: (g, 0)),
        out_shape=jax.ShapeDtypeStruct((B * N, C), x.dtype),
    )(x2, Wq, Wkv, We)

    return out.reshape(B, N, C)


# speedup vs baseline: 3.2076x; 3.2076x over previous
import jax
import jax.numpy as jnp
from jax.experimental import pallas as pl

_T = 4
_H = 12
_NB = 2


def _one_batch(xb, wq_ref, wkv_ref, we_ref):
    N, C = xb.shape
    Np = N - _T
    hd = C // _H
    TH = _T * _H
    scale = hd ** -0.5

    xt = xb[:_T, :]
    f = xb[_T:, :]
    wk = wkv_ref[:C, :]
    wv = wkv_ref[C:, :]

    q_rows = [
        jax.lax.dot_general(xt[t:t + 1, :], wq_ref[t],
                            (((1,), (1,)), ((), ())))
        for t in range(_T)
    ]
    q = jnp.concatenate(q_rows, axis=0)

    k = jax.lax.dot_general(f, wk, (((1,), (1,)), ((), ())))

    r_iota = jax.lax.broadcasted_iota(jnp.int32, (TH, C), 0)
    c_iota = jax.lax.broadcasted_iota(jnp.int32, (TH, C), 1)
    hmask = (r_iota % _H) == (c_iota // hd)

    q48 = jnp.broadcast_to(q[:, None, :], (_T, _H, C)).reshape(TH, C)
    qm = jnp.where(hmask, q48, 0.0)
    scores = jax.lax.dot_general(qm, k, (((1,), (1,)), ((), ()))) * scale

    n_iota = jax.lax.broadcasted_iota(jnp.int32, (TH, Np), 1)
    m1 = jnp.max(scores, axis=1, keepdims=True)
    idx1 = jnp.min(jnp.where(scores == m1, n_iota, Np), axis=1, keepdims=True)
    masked = jnp.where(n_iota == idx1, jnp.float32(-3.4e38), scores)
    m2 = jnp.max(masked, axis=1, keepdims=True)
    idx2 = jnp.min(jnp.where(masked == m2, n_iota, Np), axis=1, keepdims=True)

    e2 = jnp.exp(m2 - m1)
    den = 1.0 + e2
    w1 = 1.0 / den
    w2 = e2 / den

    s1 = jnp.where(n_iota == idx1, 1.0, 0.0)
    s2 = jnp.where(n_iota == idx2, 1.0, 0.0)
    d1 = s1 * w1
    d2 = s2 * w2

    g1 = jax.lax.dot_general(d1, f, (((1,), (0,)), ((), ())))
    g2 = jax.lax.dot_general(d2, f, (((1,), (0,)), ((), ())))
    gm1 = jnp.where(hmask, g1, 0.0)
    gm2 = jnp.where(hmask, g2, 0.0)

    v = jax.lax.dot_general(g1 + g2, wv, (((1,), (1,)), ((), ())))
    vm = jnp.where(hmask, v, 0.0)
    attn = vm.reshape(_T, _H, C).sum(axis=1)

    tok_rows = []
    c1_rows = []
    c2_rows = []
    for t in range(_T):
        we_t = we_ref[t]
        tok_rows.append(
            jax.lax.dot_general(attn[t:t + 1, :], we_t,
                                (((1,), (1,)), ((), ()))))
        gm_t = jnp.concatenate(
            [gm1[t * _H:(t + 1) * _H, :], gm2[t * _H:(t + 1) * _H, :]], axis=0)
        c_t = jax.lax.dot_general(gm_t, we_t, (((1,), (1,)), ((), ())))
        c1_rows.append(c_t[:_H])
        c2_rows.append(c_t[_H:])
    tok = jnp.concatenate(tok_rows, axis=0)

    c1 = jnp.concatenate(c1_rows, axis=0)
    c2 = jnp.concatenate(c2_rows, axis=0)
    feat = (jax.lax.dot_general(s1, c1, (((0,), (0,)), ((), ()))) +
            jax.lax.dot_general(s2, c2, (((0,), (0,)), ((), ()))))
    return jnp.concatenate([tok, feat], axis=0)


def _body(x_ref, wq_ref, wkv_ref, we_ref, out_ref):
    for bi in range(_NB):
        out_ref[bi] = _one_batch(x_ref[bi], wq_ref, wkv_ref, we_ref)


def kernel(x, Wq, Wkv, We):
    B, N, C = x.shape

    return pl.pallas_call(
        _body,
        grid=(B // _NB,),
        in_specs=[
            pl.BlockSpec((_NB, N, C), lambda g: (g, 0, 0)),
            pl.BlockSpec((_T, C, C), lambda g: (0, 0, 0)),
            pl.BlockSpec((2 * C, C), lambda g: (0, 0)),
            pl.BlockSpec((_T, C, C), lambda g: (0, 0, 0)),
        ],
        out_specs=pl.BlockSpec((_NB, N, C), lambda g: (g, 0, 0)),
        out_shape=jax.ShapeDtypeStruct((B, N, C), x.dtype),
    )(x, Wq, Wkv, We)
